# baseline (speedup 1.0000x reference)
import functools

import jax
import jax.numpy as jnp
from jax import lax
from jax.experimental import pallas as pl
from jax.experimental.pallas import tpu as pltpu
from jax.experimental.pallas import tpu_sc as plsc

S, N, F, C, E = 12, 10000, 128, 128, 320000
EPS = 1e-5

T1_BLK = 1000
NBLK = 400
NB = N // NBLK

CH = 50
NW = 32
EPT = E // NW
CPT = EPT // CH
NPANEL = 4
PANEL = CPT // NPANEL
NPAD = 10240


def _t1_body(x_ref, wt_ref, bt_ref, z_ref, st_ref):
    i = pl.program_id(0)
    z = jnp.dot(x_ref[...], wt_ref[...], preferred_element_type=jnp.float32)
    z = z + bt_ref[...]
    z_ref[...] = z
    s1 = jnp.sum(z, axis=0, keepdims=True)
    s2 = jnp.sum(z * z, axis=0, keepdims=True)
    st = jnp.concatenate([s1, s2], axis=0)

    @pl.when(i == 0)
    def _():
        st_ref[...] = st

    @pl.when(i > 0)
    def _():
        st_ref[...] += st


def _t1_call(xr, wt_t, bt):
    return pl.pallas_call(
        _t1_body,
        grid=(S * N // T1_BLK,),
        in_specs=[
            pl.BlockSpec((T1_BLK, F), lambda i: (i, 0)),
            pl.BlockSpec((F, C), lambda i: (0, 0)),
            pl.BlockSpec((1, C), lambda i: (0, 0)),
        ],
        out_specs=[
            pl.BlockSpec((T1_BLK, C), lambda i: (i, 0)),
            pl.BlockSpec((2, C), lambda i: (0, 0)),
        ],
        out_shape=[
            jax.ShapeDtypeStruct((S * N, C), jnp.float32),
            jax.ShapeDtypeStruct((2, C), jnp.float32),
        ],
    )(xr, wt_t, bt)


def _t2_body(z_ref, sc_ref, sh_ref, wg_ref, degt_ref, g_ref):
    zn = jnp.maximum(z_ref[0] * sc_ref[...] + sh_ref[...], 0.0)
    h = jnp.dot(zn, wg_ref[...], preferred_element_type=jnp.float32)
    dinv = lax.rsqrt(degt_ref[:, :1])
    g_ref[0] = h * dinv


def _t2_call(z3, scale, shift, wg, degt):
    return pl.pallas_call(
        _t2_body,
        grid=(S, NB),
        in_specs=[
            pl.BlockSpec((1, NBLK, C), lambda s, n: (s, n, 0)),
            pl.BlockSpec((1, C), lambda s, n: (0, 0)),
            pl.BlockSpec((1, C), lambda s, n: (0, 0)),
            pl.BlockSpec((C, C), lambda s, n: (0, 0)),
            pl.BlockSpec((NBLK, 16), lambda s, n: (n, 0)),
        ],
        out_specs=pl.BlockSpec((1, NBLK, C), lambda s, n: (s, n, 0)),
        out_shape=jax.ShapeDtypeStruct((S, N, C), jnp.float32),
    )(z3, scale, shift, wg, degt)


def _t3_body(agg_ref, g_ref, degt_ref, bg_ref, p_ref, st_ref):
    nb = pl.program_id(1)
    a = agg_ref[0, 0] + agg_ref[1, 0] + g_ref[0]
    dinv = lax.rsqrt(degt_ref[:, :1])
    p = a * dinv + bg_ref[...]
    p_ref[0] = p
    s1 = jnp.sum(p, axis=0, keepdims=True)
    s2 = jnp.sum(p * p, axis=0, keepdims=True)
    st = jnp.concatenate([s1, s2], axis=0)[None]

    @pl.when(nb == 0)
    def _():
        st_ref[...] = st

    @pl.when(nb > 0)
    def _():
        st_ref[...] += st


def _t3_call(agg, g3, degt, bg):
    return pl.pallas_call(
        _t3_body,
        grid=(S, NB),
        in_specs=[
            pl.BlockSpec((2, 1, NBLK, C), lambda s, n: (0, s, n, 0)),
            pl.BlockSpec((1, NBLK, C), lambda s, n: (s, n, 0)),
            pl.BlockSpec((NBLK, 16), lambda s, n: (n, 0)),
            pl.BlockSpec((1, C), lambda s, n: (0, 0)),
        ],
        out_specs=[
            pl.BlockSpec((1, NBLK, C), lambda s, n: (s, n, 0)),
            pl.BlockSpec((1, 2, C), lambda s, n: (s, 0, 0)),
        ],
        out_shape=[
            jax.ShapeDtypeStruct((S, N, C), jnp.float32),
            jax.ShapeDtypeStruct((S, 2, C), jnp.float32),
        ],
    )(agg, g3, degt, bg)


def _t4_body(p_ref, sc_ref, sh_ref, o_ref):
    o_ref[0] = jnp.maximum(p_ref[0] * sc_ref[0] + sh_ref[0], 0.0)


def _t4_call(p3, scale2, shift2):
    return pl.pallas_call(
        _t4_body,
        grid=(S, NB),
        in_specs=[
            pl.BlockSpec((1, NBLK, C), lambda s, n: (s, n, 0)),
            pl.BlockSpec((1, 1, C), lambda s, n: (s, 0, 0)),
            pl.BlockSpec((1, 1, C), lambda s, n: (s, 0, 0)),
        ],
        out_specs=pl.BlockSpec((1, NBLK, C), lambda s, n: (s, n, 0)),
        out_shape=jax.ShapeDtypeStruct((S, N, C), jnp.float32),
    )(p3, scale2, shift2)


NC, NS = 2, 16
STRIPE16 = NPAD // NS
_SC_MESH = dict(mesh=plsc.VectorSubcoreMesh(core_axis_name="c",
                                            subcore_axis_name="s"))


def _sc_deg_body(dst_ref, ones_ref, zeros_ref, out_ref, onesv, idxv, degsh):
    c = lax.axis_index("c")
    s = lax.axis_index("s")
    wid = c * NS + s
    pltpu.sync_copy(zeros_ref, degsh.at[pl.ds(s * STRIPE16, STRIPE16)])
    pltpu.sync_copy(ones_ref, onesv)
    plsc.subcore_barrier()

    def chunk(i, carry):
        row = wid * CPT + i
        pltpu.sync_copy(dst_ref.at[row, 0], idxv)
        pltpu.sync_copy(onesv, degsh.at[idxv], add=True)
        return carry

    lax.fori_loop(0, CPT, chunk, 0)
    plsc.subcore_barrier()
    pltpu.sync_copy(degsh.at[pl.ds(s * STRIPE16, STRIPE16)],
                    out_ref.at[c, pl.ds(s * STRIPE16, STRIPE16)])


def _sc_deg_call(dst2d, ones, zeros):
    return pl.kernel(
        _sc_deg_body,
        out_type=jax.ShapeDtypeStruct((NC, NPAD, C), jnp.float32),
        scratch_types=[
            pltpu.VMEM((CH, C), jnp.float32),
            pltpu.VMEM((CH,), jnp.int32),
            pltpu.VMEM_SHARED((NPAD, C), jnp.float32),
        ],
        **_SC_MESH,
    )(dst2d, ones, zeros)


def _sc_agg_body(g_ref, srcoff_ref, dst_ref, zeros_ref, out_ref,
                 srcv, dstv, rows0, rows1, rows2, rows3, aggsh, gsem, ssem):
    c = lax.axis_index("c")
    s = lax.axis_index("s")
    wid = c * NS + s
    bufs = (rows0, rows1, rows2, rows3)

    def per_t(t, carry):
        pltpu.sync_copy(zeros_ref, aggsh.at[pl.ds(s * STRIPE16, STRIPE16)])
        plsc.subcore_barrier()

        def panel(q, carry2):
            base = wid * CPT + q * PANEL
            pltpu.sync_copy(srcoff_ref.at[t, pl.ds(base, PANEL), 0], srcv)
            pltpu.sync_copy(dst_ref.at[pl.ds(base, PANEL), 0], dstv)
            pltpu.async_copy(g_ref.at[srcv.at[0]], rows0, gsem)
            pltpu.async_copy(g_ref.at[srcv.at[1]], rows1, gsem)

            def chunk(i, carry3):
                def _do(rbuf, nbuf):
                    pltpu.make_async_copy(g_ref.at[srcv.at[i]], rbuf,
                                          gsem).wait()
                    pltpu.async_copy(rbuf, aggsh.at[dstv.at[i]], ssem,
                                     add=True)

                    @pl.when(i >= 2)
                    def _():
                        pltpu.make_async_copy(
                            rows0, aggsh.at[dstv.at[i]], ssem).wait()

                    @pl.when(i + 2 < PANEL)
                    def _():
                        pltpu.async_copy(g_ref.at[srcv.at[i + 2]], nbuf, gsem)

                for k in range(4):
                    @pl.when(lax.rem(i, 4) == k)
                    def _(k=k):
                        _do(bufs[k], bufs[(k + 2) % 4])

                return carry3

            lax.fori_loop(0, PANEL, chunk, 0)
            pltpu.make_async_copy(rows0, aggsh.at[dstv.at[0]], ssem).wait()
            pltpu.make_async_copy(rows0, aggsh.at[dstv.at[0]], ssem).wait()
            return carry2

        lax.fori_loop(0, NPANEL, panel, 0)
        plsc.subcore_barrier()
        pltpu.sync_copy(aggsh.at[pl.ds(s * STRIPE16, STRIPE16)],
                        out_ref.at[c, t, pl.ds(s * STRIPE16, STRIPE16)])
        return carry

    lax.fori_loop(0, S, per_t, 0)


def _sc_agg_call(gflat, srcoff, dst2d, zeros):
    return pl.kernel(
        _sc_agg_body,
        out_type=jax.ShapeDtypeStruct((NC, S, NPAD, C), jnp.float32),
        scratch_types=[
            pltpu.VMEM((PANEL, CH), jnp.int32),
            pltpu.VMEM((PANEL, CH), jnp.int32),
            pltpu.VMEM((CH, C), jnp.float32),
            pltpu.VMEM((CH, C), jnp.float32),
            pltpu.VMEM((CH, C), jnp.float32),
            pltpu.VMEM((CH, C), jnp.float32),
            pltpu.VMEM_SHARED((NPAD, C), jnp.float32),
            pltpu.SemaphoreType.DMA,
            pltpu.SemaphoreType.DMA,
        ],
        **_SC_MESH,
    )(gflat, srcoff, dst2d, zeros)


def kernel(x, edge_index, Wt, bt, gamma_t, beta_t, Wg, bg, gamma_s, beta_s):
    xr = x.reshape(S * N, F)
    src = edge_index[0]
    dst = edge_index[1]
    dst2d = dst.reshape(E // CH, 1, CH)
    srcoff = (src[None] + (jnp.arange(S, dtype=jnp.int32) * N)[:, None]
              ).reshape(S, E // CH, 1, CH)
    onesC = jnp.ones((CH, C), jnp.float32)
    zerosC = jnp.zeros((STRIPE16, C), jnp.float32)

    deg2 = _sc_deg_call(dst2d, onesC, zerosC)
    degt = deg2[0, :N, :16] + deg2[1, :N, :16] + 1.0

    z, st = _t1_call(xr, Wt.T, bt[None])
    mu = st[0] / (S * N)
    var = st[1] / (S * N) - mu * mu
    rstd = lax.rsqrt(var + EPS)
    scale = (gamma_t * rstd)[None]
    shift = (beta_t - mu * rstd * gamma_t)[None]

    g3 = _t2_call(z.reshape(S, N, C), scale, shift, Wg, degt)

    agg = _sc_agg_call(g3.reshape(S * N, C), srcoff, dst2d, zerosC)

    p3, st2 = _t3_call(agg, g3, degt, bg[None])
    mu2 = st2[:, 0] / N
    var2 = st2[:, 1] / N - mu2 * mu2
    rstd2 = lax.rsqrt(var2 + EPS)
    scale2 = (gamma_s[None] * rstd2)[:, None]
    shift2 = (beta_s[None] - mu2 * rstd2 * gamma_s[None])[:, None]

    out = _t4_call(p3, scale2, shift2)
    return out.reshape(1, S, N, C)


# speedup vs baseline: 1.1321x; 1.1321x over previous
import functools

import jax
import jax.numpy as jnp
from jax import lax
from jax.experimental import pallas as pl
from jax.experimental.pallas import tpu as pltpu
from jax.experimental.pallas import tpu_sc as plsc

S, N, F, C, E = 12, 10000, 128, 128, 320000
EPS = 1e-5

T1_BLK = 1000
NBLK = 400
NB = N // NBLK

CH = 125
NW = 32
EPT = E // NW
CPT = EPT // CH
NPANEL = 2
PANEL = CPT // NPANEL
NPAD = 10240


def _t1_body(x_ref, wt_ref, bt_ref, z_ref, st_ref):
    i = pl.program_id(0)
    z = jnp.dot(x_ref[...], wt_ref[...], preferred_element_type=jnp.float32)
    z = z + bt_ref[...]
    z_ref[...] = z
    s1 = jnp.sum(z, axis=0, keepdims=True)
    s2 = jnp.sum(z * z, axis=0, keepdims=True)
    st = jnp.concatenate([s1, s2], axis=0)

    @pl.when(i == 0)
    def _():
        st_ref[...] = st

    @pl.when(i > 0)
    def _():
        st_ref[...] += st


def _t1_call(xr, wt_t, bt):
    return pl.pallas_call(
        _t1_body,
        grid=(S * N // T1_BLK,),
        in_specs=[
            pl.BlockSpec((T1_BLK, F), lambda i: (i, 0)),
            pl.BlockSpec((F, C), lambda i: (0, 0)),
            pl.BlockSpec((1, C), lambda i: (0, 0)),
        ],
        out_specs=[
            pl.BlockSpec((T1_BLK, C), lambda i: (i, 0)),
            pl.BlockSpec((2, C), lambda i: (0, 0)),
        ],
        out_shape=[
            jax.ShapeDtypeStruct((S * N, C), jnp.float32),
            jax.ShapeDtypeStruct((2, C), jnp.float32),
        ],
    )(xr, wt_t, bt)


def _t2_body(z_ref, sc_ref, sh_ref, wg_ref, degt_ref, g_ref):
    zn = jnp.maximum(z_ref[0] * sc_ref[...] + sh_ref[...], 0.0)
    h = jnp.dot(zn, wg_ref[...], preferred_element_type=jnp.float32)
    dinv = lax.rsqrt(degt_ref[:, :1])
    g_ref[0] = h * dinv


def _t2_call(z3, scale, shift, wg, degt):
    return pl.pallas_call(
        _t2_body,
        grid=(S, NB),
        in_specs=[
            pl.BlockSpec((1, NBLK, C), lambda s, n: (s, n, 0)),
            pl.BlockSpec((1, C), lambda s, n: (0, 0)),
            pl.BlockSpec((1, C), lambda s, n: (0, 0)),
            pl.BlockSpec((C, C), lambda s, n: (0, 0)),
            pl.BlockSpec((NBLK, 16), lambda s, n: (n, 0)),
        ],
        out_specs=pl.BlockSpec((1, NBLK, C), lambda s, n: (s, n, 0)),
        out_shape=jax.ShapeDtypeStruct((S, N, C), jnp.float32),
    )(z3, scale, shift, wg, degt)


def _t3_body(agg_ref, g_ref, degt_ref, bg_ref, p_ref, st_ref):
    nb = pl.program_id(1)
    a = agg_ref[0, 0] + agg_ref[1, 0] + g_ref[0]
    dinv = lax.rsqrt(degt_ref[:, :1])
    p = a * dinv + bg_ref[...]
    p_ref[0] = p
    s1 = jnp.sum(p, axis=0, keepdims=True)
    s2 = jnp.sum(p * p, axis=0, keepdims=True)
    st = jnp.concatenate([s1, s2], axis=0)[None]

    @pl.when(nb == 0)
    def _():
        st_ref[...] = st

    @pl.when(nb > 0)
    def _():
        st_ref[...] += st


def _t3_call(agg, g3, degt, bg):
    return pl.pallas_call(
        _t3_body,
        grid=(S, NB),
        in_specs=[
            pl.BlockSpec((2, 1, NBLK, C), lambda s, n: (0, s, n, 0)),
            pl.BlockSpec((1, NBLK, C), lambda s, n: (s, n, 0)),
            pl.BlockSpec((NBLK, 16), lambda s, n: (n, 0)),
            pl.BlockSpec((1, C), lambda s, n: (0, 0)),
        ],
        out_specs=[
            pl.BlockSpec((1, NBLK, C), lambda s, n: (s, n, 0)),
            pl.BlockSpec((1, 2, C), lambda s, n: (s, 0, 0)),
        ],
        out_shape=[
            jax.ShapeDtypeStruct((S, N, C), jnp.float32),
            jax.ShapeDtypeStruct((S, 2, C), jnp.float32),
        ],
    )(agg, g3, degt, bg)


def _t4_body(p_ref, sc_ref, sh_ref, o_ref):
    o_ref[0] = jnp.maximum(p_ref[0] * sc_ref[0] + sh_ref[0], 0.0)


def _t4_call(p3, scale2, shift2):
    return pl.pallas_call(
        _t4_body,
        grid=(S, NB),
        in_specs=[
            pl.BlockSpec((1, NBLK, C), lambda s, n: (s, n, 0)),
            pl.BlockSpec((1, 1, C), lambda s, n: (s, 0, 0)),
            pl.BlockSpec((1, 1, C), lambda s, n: (s, 0, 0)),
        ],
        out_specs=pl.BlockSpec((1, NBLK, C), lambda s, n: (s, n, 0)),
        out_shape=jax.ShapeDtypeStruct((S, N, C), jnp.float32),
    )(p3, scale2, shift2)


NC, NS = 2, 16
STRIPE16 = NPAD // NS
_SC_MESH = dict(mesh=plsc.VectorSubcoreMesh(core_axis_name="c",
                                            subcore_axis_name="s"))


def _sc_deg_body(dst_ref, ones_ref, zeros_ref, out_ref, onesv, idxv, degsh):
    c = lax.axis_index("c")
    s = lax.axis_index("s")
    wid = c * NS + s
    pltpu.sync_copy(zeros_ref, degsh.at[pl.ds(s * STRIPE16, STRIPE16)])
    pltpu.sync_copy(ones_ref, onesv)
    plsc.subcore_barrier()

    def chunk(i, carry):
        row = wid * CPT + i
        pltpu.sync_copy(dst_ref.at[row, 0], idxv)
        pltpu.sync_copy(onesv, degsh.at[idxv], add=True)
        return carry

    lax.fori_loop(0, CPT, chunk, 0)
    plsc.subcore_barrier()
    pltpu.sync_copy(degsh.at[pl.ds(s * STRIPE16, STRIPE16)],
                    out_ref.at[c, pl.ds(s * STRIPE16, STRIPE16)])


def _sc_deg_call(dst2d, ones, zeros):
    return pl.kernel(
        _sc_deg_body,
        out_type=jax.ShapeDtypeStruct((NC, NPAD, C), jnp.float32),
        scratch_types=[
            pltpu.VMEM((CH, C), jnp.float32),
            pltpu.VMEM((CH,), jnp.int32),
            pltpu.VMEM_SHARED((NPAD, C), jnp.float32),
        ],
        **_SC_MESH,
    )(dst2d, ones, zeros)


def _sc_agg_body(g_ref, srcoff_ref, dst_ref, zeros_ref, out_ref,
                 srcv, dstv, rows0, rows1, aggsh, gsem):
    c = lax.axis_index("c")
    s = lax.axis_index("s")
    wid = c * NS + s

    def per_t(t, carry):
        pltpu.sync_copy(zeros_ref, aggsh.at[pl.ds(s * STRIPE16, STRIPE16)])
        plsc.subcore_barrier()

        def panel(q, carry2):
            base = wid * CPT + q * PANEL
            pltpu.sync_copy(srcoff_ref.at[t, pl.ds(base, PANEL), 0], srcv)
            pltpu.sync_copy(dst_ref.at[pl.ds(base, PANEL), 0], dstv)
            pltpu.async_copy(g_ref.at[srcv.at[0]], rows0, gsem)
            pltpu.async_copy(g_ref.at[srcv.at[1]], rows1, gsem)

            def chunk(i, carry3):
                def _do(rbuf):
                    pltpu.make_async_copy(g_ref.at[srcv.at[i]], rbuf,
                                          gsem).wait()
                    pltpu.sync_copy(rbuf, aggsh.at[dstv.at[i]], add=True)

                    @pl.when(i + 2 < PANEL)
                    def _():
                        pltpu.async_copy(g_ref.at[srcv.at[i + 2]], rbuf, gsem)

                @pl.when(lax.rem(i, 2) == 0)
                def _():
                    _do(rows0)

                @pl.when(lax.rem(i, 2) == 1)
                def _():
                    _do(rows1)

                return carry3

            lax.fori_loop(0, PANEL, chunk, 0)
            return carry2

        lax.fori_loop(0, NPANEL, panel, 0)
        plsc.subcore_barrier()
        pltpu.sync_copy(aggsh.at[pl.ds(s * STRIPE16, STRIPE16)],
                        out_ref.at[c, t, pl.ds(s * STRIPE16, STRIPE16)])
        return carry

    lax.fori_loop(0, S, per_t, 0)


def _sc_agg_call(gflat, srcoff, dst2d, zeros):
    return pl.kernel(
        _sc_agg_body,
        out_type=jax.ShapeDtypeStruct((NC, S, NPAD, C), jnp.float32),
        scratch_types=[
            pltpu.VMEM((PANEL, CH), jnp.int32),
            pltpu.VMEM((PANEL, CH), jnp.int32),
            pltpu.VMEM((CH, C), jnp.float32),
            pltpu.VMEM((CH, C), jnp.float32),
            pltpu.VMEM_SHARED((NPAD, C), jnp.float32),
            pltpu.SemaphoreType.DMA,
        ],
        **_SC_MESH,
    )(gflat, srcoff, dst2d, zeros)


def kernel(x, edge_index, Wt, bt, gamma_t, beta_t, Wg, bg, gamma_s, beta_s):
    xr = x.reshape(S * N, F)
    src = edge_index[0]
    dst = edge_index[1]
    dst2d = dst.reshape(E // CH, 1, CH)
    srcoff = (src[None] + (jnp.arange(S, dtype=jnp.int32) * N)[:, None]
              ).reshape(S, E // CH, 1, CH)
    onesC = jnp.ones((CH, C), jnp.float32)
    zerosC = jnp.zeros((STRIPE16, C), jnp.float32)

    deg2 = _sc_deg_call(dst2d, onesC, zerosC)
    degt = deg2[0, :N, :16] + deg2[1, :N, :16] + 1.0

    z, st = _t1_call(xr, Wt.T, bt[None])
    mu = st[0] / (S * N)
    var = st[1] / (S * N) - mu * mu
    rstd = lax.rsqrt(var + EPS)
    scale = (gamma_t * rstd)[None]
    shift = (beta_t - mu * rstd * gamma_t)[None]

    g3 = _t2_call(z.reshape(S, N, C), scale, shift, Wg, degt)

    agg = _sc_agg_call(g3.reshape(S * N, C), srcoff, dst2d, zerosC)

    p3, st2 = _t3_call(agg, g3, degt, bg[None])
    mu2 = st2[:, 0] / N
    var2 = st2[:, 1] / N - mu2 * mu2
    rstd2 = lax.rsqrt(var2 + EPS)
    scale2 = (gamma_s[None] * rstd2)[:, None]
    shift2 = (beta_s[None] - mu2 * rstd2 * gamma_s[None])[:, None]

    out = _t4_call(p3, scale2, shift2)
    return out.reshape(1, S, N, C)


# speedup vs baseline: 1.1625x; 1.0268x over previous
import functools

import jax
import jax.numpy as jnp
from jax import lax
from jax.experimental import pallas as pl
from jax.experimental.pallas import tpu as pltpu
from jax.experimental.pallas import tpu_sc as plsc

S, N, F, C, E = 12, 10000, 128, 128, 320000
EPS = 1e-5

T1_BLK = 1000
NBLK = 400
NB = N // NBLK

CH = 100
NW = 32
EPT = E // NW
CPT = EPT // CH
NPANEL = 5
PANEL = CPT // NPANEL
ASTRIDE = 624
NPAD = 10240


def _t1_body(x_ref, wt_ref, bt_ref, z_ref, st_ref):
    i = pl.program_id(0)
    z = jnp.dot(x_ref[...], wt_ref[...], preferred_element_type=jnp.float32)
    z = z + bt_ref[...]
    z_ref[...] = z
    s1 = jnp.sum(z, axis=0, keepdims=True)
    s2 = jnp.sum(z * z, axis=0, keepdims=True)
    st = jnp.concatenate([s1, s2], axis=0)

    @pl.when(i == 0)
    def _():
        st_ref[...] = st

    @pl.when(i > 0)
    def _():
        st_ref[...] += st


def _t1_call(xr, wt_t, bt):
    return pl.pallas_call(
        _t1_body,
        grid=(S * N // T1_BLK,),
        in_specs=[
            pl.BlockSpec((T1_BLK, F), lambda i: (i, 0)),
            pl.BlockSpec((F, C), lambda i: (0, 0)),
            pl.BlockSpec((1, C), lambda i: (0, 0)),
        ],
        out_specs=[
            pl.BlockSpec((T1_BLK, C), lambda i: (i, 0)),
            pl.BlockSpec((2, C), lambda i: (0, 0)),
        ],
        out_shape=[
            jax.ShapeDtypeStruct((S * N, C), jnp.float32),
            jax.ShapeDtypeStruct((2, C), jnp.float32),
        ],
    )(xr, wt_t, bt)


def _t2_body(z_ref, sc_ref, sh_ref, wg_ref, degt_ref, g_ref):
    zn = jnp.maximum(z_ref[0] * sc_ref[...] + sh_ref[...], 0.0)
    h = jnp.dot(zn, wg_ref[...], preferred_element_type=jnp.float32)
    dinv = lax.rsqrt(degt_ref[:, :1])
    g_ref[0] = h * dinv


def _t2_call(z3, scale, shift, wg, degt):
    return pl.pallas_call(
        _t2_body,
        grid=(S, NB),
        in_specs=[
            pl.BlockSpec((1, NBLK, C), lambda s, n: (s, n, 0)),
            pl.BlockSpec((1, C), lambda s, n: (0, 0)),
            pl.BlockSpec((1, C), lambda s, n: (0, 0)),
            pl.BlockSpec((C, C), lambda s, n: (0, 0)),
            pl.BlockSpec((NBLK, 16), lambda s, n: (n, 0)),
        ],
        out_specs=pl.BlockSpec((1, NBLK, C), lambda s, n: (s, n, 0)),
        out_shape=jax.ShapeDtypeStruct((S, N, C), jnp.float32),
    )(z3, scale, shift, wg, degt)


def _t3_body(agg_ref, g_ref, degt_ref, bg_ref, p_ref, st_ref):
    nb = pl.program_id(1)
    a = agg_ref[0, 0] + agg_ref[1, 0] + g_ref[0]
    dinv = lax.rsqrt(degt_ref[:, :1])
    p = a * dinv + bg_ref[...]
    p_ref[0] = p
    s1 = jnp.sum(p, axis=0, keepdims=True)
    s2 = jnp.sum(p * p, axis=0, keepdims=True)
    st = jnp.concatenate([s1, s2], axis=0)[None]

    @pl.when(nb == 0)
    def _():
        st_ref[...] = st

    @pl.when(nb > 0)
    def _():
        st_ref[...] += st


def _t3_call(agg, g3, degt, bg):
    return pl.pallas_call(
        _t3_body,
        grid=(S, NB),
        in_specs=[
            pl.BlockSpec((2, 1, NBLK, C), lambda s, n: (0, s, n, 0)),
            pl.BlockSpec((1, NBLK, C), lambda s, n: (s, n, 0)),
            pl.BlockSpec((NBLK, 16), lambda s, n: (n, 0)),
            pl.BlockSpec((1, C), lambda s, n: (0, 0)),
        ],
        out_specs=[
            pl.BlockSpec((1, NBLK, C), lambda s, n: (s, n, 0)),
            pl.BlockSpec((1, 2, C), lambda s, n: (s, 0, 0)),
        ],
        out_shape=[
            jax.ShapeDtypeStruct((S, N, C), jnp.float32),
            jax.ShapeDtypeStruct((S, 2, C), jnp.float32),
        ],
    )(agg, g3, degt, bg)


def _t4_body(p_ref, sc_ref, sh_ref, o_ref):
    o_ref[0] = jnp.maximum(p_ref[0] * sc_ref[0] + sh_ref[0], 0.0)


def _t4_call(p3, scale2, shift2):
    return pl.pallas_call(
        _t4_body,
        grid=(S, NB),
        in_specs=[
            pl.BlockSpec((1, NBLK, C), lambda s, n: (s, n, 0)),
            pl.BlockSpec((1, 1, C), lambda s, n: (s, 0, 0)),
            pl.BlockSpec((1, 1, C), lambda s, n: (s, 0, 0)),
        ],
        out_specs=pl.BlockSpec((1, NBLK, C), lambda s, n: (s, n, 0)),
        out_shape=jax.ShapeDtypeStruct((S, N, C), jnp.float32),
    )(p3, scale2, shift2)


NC, NS = 2, 16
STRIPE16 = NPAD // NS
_SC_MESH = dict(mesh=plsc.VectorSubcoreMesh(core_axis_name="c",
                                            subcore_axis_name="s"))


def _sc_deg_body(dst_ref, ones_ref, zeros_ref, out_ref, onesv, idxv, degsh):
    c = lax.axis_index("c")
    s = lax.axis_index("s")
    wid = c * NS + s
    pltpu.sync_copy(zeros_ref, degsh.at[pl.ds(s * STRIPE16, STRIPE16)])
    pltpu.sync_copy(ones_ref, onesv)
    plsc.subcore_barrier()

    def chunk(i, carry):
        row = wid * CPT + i
        pltpu.sync_copy(dst_ref.at[row, 0], idxv)
        pltpu.sync_copy(onesv, degsh.at[idxv], add=True)
        return carry

    lax.fori_loop(0, CPT, chunk, 0)
    plsc.subcore_barrier()
    pltpu.sync_copy(degsh.at[pl.ds(s * STRIPE16, STRIPE16)],
                    out_ref.at[c, pl.ds(s * STRIPE16, STRIPE16)])


def _sc_deg_call(dst2d, ones, zeros):
    return pl.kernel(
        _sc_deg_body,
        out_type=jax.ShapeDtypeStruct((NC, NPAD, C), jnp.float32),
        scratch_types=[
            pltpu.VMEM((CH, C), jnp.float32),
            pltpu.VMEM((CH,), jnp.int32),
            pltpu.VMEM_SHARED((NPAD, C), jnp.float32),
        ],
        **_SC_MESH,
    )(dst2d, ones, zeros)


def _sc_agg_body(g_ref, srcoff_ref, dst_ref, zeros_ref, out_ref,
                 srcv, dstv, rows0, rows1, rows2, aggsh, gsem, ssem):
    c = lax.axis_index("c")
    s = lax.axis_index("s")
    wid = c * NS + s

    bufs = (rows0, rows1, rows2)

    def per_t(t, carry):
        pltpu.sync_copy(zeros_ref, aggsh.at[pl.ds(s * ASTRIDE, STRIPE16)])
        plsc.subcore_barrier()

        def panel(q, carry2):
            base = wid * CPT + q * PANEL
            pltpu.sync_copy(srcoff_ref.at[t, pl.ds(base, PANEL), 0], srcv)
            pltpu.sync_copy(dst_ref.at[pl.ds(base, PANEL), 0], dstv)
            pltpu.async_copy(g_ref.at[srcv.at[0]], rows0, gsem)
            pltpu.async_copy(g_ref.at[srcv.at[1]], rows1, gsem)

            def chunk(i, carry3):
                def _do(rbuf, nbuf):
                    pltpu.make_async_copy(g_ref.at[srcv.at[i]], rbuf,
                                          gsem).wait()
                    pltpu.async_copy(rbuf, aggsh.at[dstv.at[i]], ssem,
                                     add=True)

                    @pl.when(i >= 1)
                    def _():
                        pltpu.make_async_copy(
                            rows0, aggsh.at[dstv.at[i]], ssem).wait()

                    @pl.when(i + 2 < PANEL)
                    def _():
                        pltpu.async_copy(g_ref.at[srcv.at[i + 2]], nbuf, gsem)

                for k in range(3):
                    @pl.when(lax.rem(i, 3) == k)
                    def _(k=k):
                        _do(bufs[k], bufs[(k + 2) % 3])

                return carry3

            lax.fori_loop(0, PANEL, chunk, 0)
            pltpu.make_async_copy(rows0, aggsh.at[dstv.at[0]], ssem).wait()
            return carry2

        lax.fori_loop(0, NPANEL, panel, 0)
        plsc.subcore_barrier()
        pltpu.sync_copy(aggsh.at[pl.ds(s * ASTRIDE, STRIPE16)],
                        out_ref.at[c, t, pl.ds(s * ASTRIDE, STRIPE16)])
        return carry

    lax.fori_loop(0, S, per_t, 0)


def _sc_agg_call(gflat, srcoff, dst2d, zeros):
    return pl.kernel(
        _sc_agg_body,
        out_type=jax.ShapeDtypeStruct((NC, S, N, C), jnp.float32),
        scratch_types=[
            pltpu.VMEM((PANEL, CH), jnp.int32),
            pltpu.VMEM((PANEL, CH), jnp.int32),
            pltpu.VMEM((CH, C), jnp.float32),
            pltpu.VMEM((CH, C), jnp.float32),
            pltpu.VMEM((CH, C), jnp.float32),
            pltpu.VMEM_SHARED((N, C), jnp.float32),
            pltpu.SemaphoreType.DMA,
            pltpu.SemaphoreType.DMA,
        ],
        **_SC_MESH,
    )(gflat, srcoff, dst2d, zeros)


def kernel(x, edge_index, Wt, bt, gamma_t, beta_t, Wg, bg, gamma_s, beta_s):
    xr = x.reshape(S * N, F)
    src = edge_index[0]
    dst = edge_index[1]
    dst2d = dst.reshape(E // CH, 1, CH)
    srcoff = (src[None] + (jnp.arange(S, dtype=jnp.int32) * N)[:, None]
              ).reshape(S, E // CH, 1, CH)
    onesC = jnp.ones((CH, C), jnp.float32)
    zerosC = jnp.zeros((STRIPE16, C), jnp.float32)

    deg2 = _sc_deg_call(dst2d, onesC, zerosC)
    degt = deg2[0, :N, :16] + deg2[1, :N, :16] + 1.0

    z, st = _t1_call(xr, Wt.T, bt[None])
    mu = st[0] / (S * N)
    var = st[1] / (S * N) - mu * mu
    rstd = lax.rsqrt(var + EPS)
    scale = (gamma_t * rstd)[None]
    shift = (beta_t - mu * rstd * gamma_t)[None]

    g3 = _t2_call(z.reshape(S, N, C), scale, shift, Wg, degt)

    agg = _sc_agg_call(g3.reshape(S * N, C), srcoff, dst2d, zerosC)

    p3, st2 = _t3_call(agg, g3, degt, bg[None])
    mu2 = st2[:, 0] / N
    var2 = st2[:, 1] / N - mu2 * mu2
    rstd2 = lax.rsqrt(var2 + EPS)
    scale2 = (gamma_s[None] * rstd2)[:, None]
    shift2 = (beta_s[None] - mu2 * rstd2 * gamma_s[None])[:, None]

    out = _t4_call(p3, scale2, shift2)
    return out.reshape(1, S, N, C)
